# baseline (speedup 1.0000x reference)
import functools

import jax
import jax.numpy as jnp
from jax import lax
from jax.experimental import pallas as pl
from jax.experimental.pallas import tpu as pltpu
from jax.experimental.pallas import tpu_sc as plsc

N_ASPECTS = 100000
N_CLASSES = 4
D = 128
B = 4096
H = 50

NC = 2
NS = 16
NW = NC * NS
RPW = B // NW
CH = 8
NCHUNK = RPW // CH
IDS_PER_CHUNK = CH * H
SUB = 80
NSUB = IDS_PER_CHUNK // SUB
NVD = D // 16


def _body(aspect_hbm, class_hbm, table_hbm, out_hbm,
          av, cv, cntv, t0, rows0, rows1, ob0, ob1,
          gsem0, gsem1, osem0, osem1):
    wid = lax.axis_index("s") * NC + lax.axis_index("c")
    base = wid * RPW
    fbase = base * H

    pltpu.sync_copy(aspect_hbm.at[pl.ds(fbase, RPW * H)], av)
    pltpu.sync_copy(class_hbm.at[pl.ds(fbase, RPW * H)], cv)
    pltpu.sync_copy(table_hbm.at[pl.ds(0, 1)], t0)

    @pl.loop(0, RPW * H // 16, unroll=4)
    def _ids(i):
        off = pl.multiple_of(i * 16, 16)
        a = av[pl.ds(off, 16)]
        c = cv[pl.ds(off, 16)]
        idv = a * N_CLASSES + c
        av[pl.ds(off, 16)] = jnp.where(idv != 0, idv + 1, idv)

    lanes = lax.iota(jnp.int32, 16)

    def fire(c, rbuf, sem):
        for j in range(NSUB):
            off = pl.multiple_of(c * IDS_PER_CHUNK + j * SUB, 8)
            pltpu.async_copy(
                table_hbm.at[av.at[pl.ds(off, SUB)]],
                rbuf.at[pl.ds(j * SUB, SUB)], sem)

    def drain(rbuf, sem):
        for j in range(NSUB):
            pltpu.make_async_copy(
                table_hbm.at[av.at[pl.ds(j * SUB, SUB)]],
                rbuf.at[pl.ds(j * SUB, SUB)], sem).wait()

    def accum(c, rbuf, obuf):
        for b in range(CH):
            r = c * CH + b
            goff = pl.multiple_of((r // 16) * 16, 16)
            cgrp = cntv[pl.ds(goff, 16)]
            pad = jnp.sum(jnp.where(lanes == r % 16, cgrp, 0.0))
            padv = jnp.broadcast_to(pad, (16,))
            invv = 1.0 / (jnp.float32(H) - padv)

            def abody(l, accs):
                row = b * H + l * 2
                return tuple(
                    accs[v] + rbuf[row, pl.ds(v * 16, 16)]
                    + rbuf[row + 1, pl.ds(v * 16, 16)]
                    for v in range(NVD))

            accs = lax.fori_loop(
                0, H // 2, abody,
                tuple(jnp.zeros((16,), jnp.float32) for _ in range(NVD)))
            for v in range(NVD):
                t0s = t0[0, pl.ds(v * 16, 16)]
                obuf[b, pl.ds(v * 16, 16)] = (accs[v] - padv * t0s) * invv

    def out_dma(c, obuf, osem):
        ooff = pl.multiple_of(base + c * CH, 8)
        pltpu.async_copy(obuf, out_hbm.at[pl.ds(ooff, CH)], osem)

    def out_wait(obuf, osem):
        pltpu.make_async_copy(obuf, out_hbm.at[pl.ds(0, CH)], osem).wait()

    fire(0, rows0, gsem0)
    fire(1, rows1, gsem1)

    @pl.loop(0, RPW // 16)
    def _cnt(g):
        rowv = g * 16 + lanes

        def cbody(l, acc):
            idv = plsc.load_gather(av, [rowv * H + l])
            return acc + jnp.where(idv == 0, 1.0, 0.0).astype(jnp.float32)

        acc = lax.fori_loop(0, H, cbody, jnp.zeros((16,), jnp.float32))
        cntv[pl.ds(pl.multiple_of(g * 16, 16), 16)] = acc

    drain(rows0, gsem0)
    accum(0, rows0, ob0)
    fire(2, rows0, gsem0)
    out_dma(0, ob0, osem0)
    drain(rows1, gsem1)
    accum(1, rows1, ob1)
    fire(3, rows1, gsem1)
    out_dma(1, ob1, osem1)

    @pl.loop(1, NCHUNK // 2 - 1)
    def _steps(s):
        c0 = s * 2
        c1 = c0 + 1
        drain(rows0, gsem0)
        out_wait(ob0, osem0)
        accum(c0, rows0, ob0)
        fire(c0 + 2, rows0, gsem0)
        out_dma(c0, ob0, osem0)
        drain(rows1, gsem1)
        out_wait(ob1, osem1)
        accum(c1, rows1, ob1)
        fire(c1 + 2, rows1, gsem1)
        out_dma(c1, ob1, osem1)

    drain(rows0, gsem0)
    out_wait(ob0, osem0)
    accum(NCHUNK - 2, rows0, ob0)
    out_dma(NCHUNK - 2, ob0, osem0)
    drain(rows1, gsem1)
    out_wait(ob1, osem1)
    accum(NCHUNK - 1, rows1, ob1)
    out_dma(NCHUNK - 1, ob1, osem1)
    out_wait(ob0, osem0)
    out_wait(ob1, osem1)


@jax.jit
def _run(aspect_flat, class_flat, table):
    mesh = plsc.VectorSubcoreMesh(core_axis_name="c", subcore_axis_name="s")
    k = pl.kernel(
        _body,
        out_type=jax.ShapeDtypeStruct((B, D), jnp.float32),
        mesh=mesh,
        compiler_params=pltpu.CompilerParams(needs_layout_passes=False),
        scratch_types=[
            pltpu.VMEM((RPW * H,), jnp.int32),
            pltpu.VMEM((RPW * H,), jnp.int32),
            pltpu.VMEM((RPW,), jnp.float32),
            pltpu.VMEM((1, D), jnp.float32),
            pltpu.VMEM((IDS_PER_CHUNK, D), jnp.float32),
            pltpu.VMEM((IDS_PER_CHUNK, D), jnp.float32),
            pltpu.VMEM((CH, D), jnp.float32),
            pltpu.VMEM((CH, D), jnp.float32),
            pltpu.SemaphoreType.DMA,
            pltpu.SemaphoreType.DMA,
            pltpu.SemaphoreType.DMA,
            pltpu.SemaphoreType.DMA,
        ],
    )
    return k(aspect_flat, class_flat, table)


def kernel(aspect_ids, class_ids, table):
    aspect_flat = aspect_ids.astype(jnp.int32).reshape(-1)
    class_flat = class_ids.astype(jnp.int32).reshape(-1)
    return _run(aspect_flat, class_flat, table.astype(jnp.float32))


# speedup vs baseline: 1.0594x; 1.0594x over previous
import jax
import jax.numpy as jnp
from jax import lax
from jax.experimental import pallas as pl
from jax.experimental.pallas import tpu as pltpu
from jax.experimental.pallas import tpu_sc as plsc

N_CLASSES = 4
D = 128
B = 4096
H = 50

NC = 2
NS = 16
NW = NC * NS
RPW = B // NW
CH = 4
NCHUNK = RPW // CH
IPC = CH * H
SUBS = ((0, 104), (104, 96))
NVD = D // 16
DEPTH = 4
NITER = NCHUNK // DEPTH


def _body(aspect_hbm, class_hbm, table_hbm, out_hbm,
          av, cv, cntv, t0, rows0, rows1, rows2, rows3, ob0, ob1,
          gsem0, gsem1, gsem2, gsem3, osem0, osem1):
    rows = (rows0, rows1, rows2, rows3)
    gsem = (gsem0, gsem1, gsem2, gsem3)
    obs = (ob0, ob1)
    osem = (osem0, osem1)

    wid = lax.axis_index("s") * NC + lax.axis_index("c")
    base = wid * RPW
    fbase = base * H

    pltpu.sync_copy(aspect_hbm.at[pl.ds(fbase, RPW * H)], av)
    pltpu.sync_copy(class_hbm.at[pl.ds(fbase, RPW * H)], cv)
    pltpu.sync_copy(table_hbm.at[pl.ds(0, 1)], t0)

    @pl.loop(0, RPW * H // 16, unroll=4)
    def _ids(i):
        off = pl.multiple_of(i * 16, 16)
        a = av[pl.ds(off, 16)]
        c = cv[pl.ds(off, 16)]
        idv = a * N_CLASSES + c
        av[pl.ds(off, 16)] = jnp.where(idv != 0, idv + 1, idv)

    lanes = lax.iota(jnp.int32, 16)

    def fire(c, rbuf, sem):
        for soff, slen in SUBS:
            off = pl.multiple_of(c * IPC + soff, 8)
            pltpu.async_copy(
                table_hbm.at[av.at[pl.ds(off, slen)]],
                rbuf.at[pl.ds(soff, slen)], sem)

    def drain(rbuf, sem):
        for soff, slen in SUBS:
            pltpu.make_async_copy(
                table_hbm.at[av.at[pl.ds(soff, slen)]],
                rbuf.at[pl.ds(soff, slen)], sem).wait()

    def accum(c, rbuf, obuf, ohalf):
        for b in range(CH):
            r = c * CH + b
            goff = pl.multiple_of((r // 16) * 16, 16)
            cgrp = cntv[pl.ds(goff, 16)]
            pad = jnp.sum(jnp.where(lanes == r % 16, cgrp, 0.0))
            padv = jnp.broadcast_to(pad, (16,))
            invv = 1.0 / (jnp.float32(H) - padv)

            def abody(l, accs):
                row = b * H + l * 2
                return tuple(
                    accs[v] + rbuf[row, pl.ds(v * 16, 16)]
                    + rbuf[row + 1, pl.ds(v * 16, 16)]
                    for v in range(NVD))

            accs = lax.fori_loop(
                0, H // 2, abody,
                tuple(jnp.zeros((16,), jnp.float32) for _ in range(NVD)))
            for v in range(NVD):
                t0s = t0[0, pl.ds(v * 16, 16)]
                obuf[ohalf * CH + b, pl.ds(v * 16, 16)] = \
                    (accs[v] - padv * t0s) * invv

    def out_dma(cpair, obuf, sem):
        ooff = pl.multiple_of(base + cpair * 2 * CH, 8)
        pltpu.async_copy(obuf, out_hbm.at[pl.ds(ooff, 2 * CH)], sem)

    def out_wait(obuf, sem):
        pltpu.make_async_copy(obuf, out_hbm.at[pl.ds(0, 2 * CH)], sem).wait()

    for d in range(DEPTH):
        fire(d, rows[d], gsem[d])

    @pl.loop(0, RPW // 16)
    def _cnt(g):
        rowv = g * 16 + lanes

        def cbody(l, acc):
            idv = plsc.load_gather(av, [rowv * H + l])
            return acc + jnp.where(idv == 0, 1.0, 0.0).astype(jnp.float32)

        acc = lax.fori_loop(0, H, cbody, jnp.zeros((16,), jnp.float32))
        cntv[pl.ds(pl.multiple_of(g * 16, 16), 16)] = acc

    def ring_step(s, first, last):
        c0 = s * DEPTH
        for d in range(DEPTH):
            c = c0 + d
            drain(rows[d], gsem[d])
            if not first and d < 2:
                out_wait(obs[d], osem[d])
            accum(c, rows[d], obs[d // 2], d % 2)
            if not last:
                fire(c + DEPTH, rows[d], gsem[d])
            if d % 2 == 1:
                out_dma(s * 2 + d // 2, obs[d // 2], osem[d // 2])

    ring_step(0, True, False)

    @pl.loop(1, NITER - 1)
    def _steps(s):
        ring_step(s, False, False)

    ring_step(NITER - 1, False, True)
    out_wait(ob0, osem0)
    out_wait(ob1, osem1)


@jax.jit
def _run(aspect_flat, class_flat, table):
    mesh = plsc.VectorSubcoreMesh(core_axis_name="c", subcore_axis_name="s")
    k = pl.kernel(
        _body,
        out_type=jax.ShapeDtypeStruct((B, D), jnp.float32),
        mesh=mesh,
        compiler_params=pltpu.CompilerParams(needs_layout_passes=False),
        scratch_types=[
            pltpu.VMEM((RPW * H,), jnp.int32),
            pltpu.VMEM((RPW * H,), jnp.int32),
            pltpu.VMEM((RPW,), jnp.float32),
            pltpu.VMEM((1, D), jnp.float32),
            pltpu.VMEM((IPC, D), jnp.float32),
            pltpu.VMEM((IPC, D), jnp.float32),
            pltpu.VMEM((IPC, D), jnp.float32),
            pltpu.VMEM((IPC, D), jnp.float32),
            pltpu.VMEM((2 * CH, D), jnp.float32),
            pltpu.VMEM((2 * CH, D), jnp.float32),
            pltpu.SemaphoreType.DMA,
            pltpu.SemaphoreType.DMA,
            pltpu.SemaphoreType.DMA,
            pltpu.SemaphoreType.DMA,
            pltpu.SemaphoreType.DMA,
            pltpu.SemaphoreType.DMA,
        ],
    )
    return k(aspect_flat, class_flat, table)


def kernel(aspect_ids, class_ids, table):
    aspect_flat = aspect_ids.astype(jnp.int32).reshape(-1)
    class_flat = class_ids.astype(jnp.int32).reshape(-1)
    return _run(aspect_flat, class_flat, table.astype(jnp.float32))


# speedup vs baseline: 1.0767x; 1.0163x over previous
import jax
import jax.numpy as jnp
from jax import lax
from jax.experimental import pallas as pl
from jax.experimental.pallas import tpu as pltpu
from jax.experimental.pallas import tpu_sc as plsc

N_CLASSES = 4
D = 128
B = 4096
H = 50

NC = 2
NS = 16
NW = NC * NS
RPW = B // NW
CH = 4
NCHUNK = RPW // CH
IPC = CH * H
SUBS = ((0, 104), (104, 96))
NVD = D // 16
DEPTH = 4
NITER = NCHUNK // DEPTH


def _body(aspect_hbm, class_hbm, table_hbm, out_hbm,
          av, cv, cntv, t0, rows0, rows1, rows2, rows3, ob0, ob1,
          g0a, g0b, g1a, g1b, g2a, g2b, g3a, g3b, osem0, osem1):
    rows = (rows0, rows1, rows2, rows3)
    gsem = ((g0a, g0b), (g1a, g1b), (g2a, g2b), (g3a, g3b))
    obs = (ob0, ob1)
    osem = (osem0, osem1)

    wid = lax.axis_index("s") * NC + lax.axis_index("c")
    base = wid * RPW
    fbase = base * H

    pltpu.sync_copy(aspect_hbm.at[pl.ds(fbase, RPW * H)], av)
    pltpu.sync_copy(class_hbm.at[pl.ds(fbase, RPW * H)], cv)
    pltpu.sync_copy(table_hbm.at[pl.ds(0, 1)], t0)

    @pl.loop(0, RPW * H // 16, unroll=4)
    def _ids(i):
        off = pl.multiple_of(i * 16, 16)
        a = av[pl.ds(off, 16)]
        c = cv[pl.ds(off, 16)]
        idv = a * N_CLASSES + c
        av[pl.ds(off, 16)] = jnp.where(idv != 0, idv + 1, idv)

    lanes = lax.iota(jnp.int32, 16)

    def fire(c, rbuf, sems):
        for (soff, slen), sem in zip(SUBS, sems):
            off = pl.multiple_of(c * IPC + soff, 8)
            pltpu.async_copy(
                table_hbm.at[av.at[pl.ds(off, slen)]],
                rbuf.at[pl.ds(soff, slen)], sem)

    def drain_sub(rbuf, sems, j):
        soff, slen = SUBS[j]
        pltpu.make_async_copy(
            table_hbm.at[av.at[pl.ds(soff, slen)]],
            rbuf.at[pl.ds(soff, slen)], sems[j]).wait()

    def accum(c, rbuf, obuf, ohalf, b_lo, b_hi):
        for b in range(b_lo, b_hi):
            r = c * CH + b
            goff = pl.multiple_of((r // 16) * 16, 16)
            cgrp = cntv[pl.ds(goff, 16)]
            pad = jnp.sum(jnp.where(lanes == r % 16, cgrp, 0.0))
            padv = jnp.broadcast_to(pad, (16,))
            invv = 1.0 / (jnp.float32(H) - padv)

            def abody(l, accs):
                row = b * H + l * 2
                return tuple(
                    accs[v] + rbuf[row, pl.ds(v * 16, 16)]
                    + rbuf[row + 1, pl.ds(v * 16, 16)]
                    for v in range(NVD))

            accs = lax.fori_loop(
                0, H // 2, abody,
                tuple(jnp.zeros((16,), jnp.float32) for _ in range(NVD)))
            for v in range(NVD):
                t0s = t0[0, pl.ds(v * 16, 16)]
                obuf[ohalf * CH + b, pl.ds(v * 16, 16)] = \
                    (accs[v] - padv * t0s) * invv

    def out_dma(cpair, obuf, sem):
        ooff = pl.multiple_of(base + cpair * 2 * CH, 8)
        pltpu.async_copy(obuf, out_hbm.at[pl.ds(ooff, 2 * CH)], sem)

    def out_wait(obuf, sem):
        pltpu.make_async_copy(obuf, out_hbm.at[pl.ds(0, 2 * CH)], sem).wait()

    for d in range(DEPTH):
        fire(d, rows[d], gsem[d])

    @pl.loop(0, RPW // 16)
    def _cnt(g):
        rowv = g * 16 + lanes

        def cbody(l, acc):
            idv = plsc.load_gather(av, [rowv * H + l])
            return acc + jnp.where(idv == 0, 1.0, 0.0).astype(jnp.float32)

        acc = lax.fori_loop(0, H, cbody, jnp.zeros((16,), jnp.float32))
        cntv[pl.ds(pl.multiple_of(g * 16, 16), 16)] = acc

    def ring_step(s, first, last):
        c0 = s * DEPTH
        for d in range(DEPTH):
            c = c0 + d
            drain_sub(rows[d], gsem[d], 0)
            if not first and d < 2:
                out_wait(obs[d], osem[d])
            accum(c, rows[d], obs[d // 2], d % 2, 0, 2)
            drain_sub(rows[d], gsem[d], 1)
            accum(c, rows[d], obs[d // 2], d % 2, 2, CH)
            if not last:
                fire(c + DEPTH, rows[d], gsem[d])
            if d % 2 == 1:
                out_dma(s * 2 + d // 2, obs[d // 2], osem[d // 2])

    ring_step(0, True, False)

    @pl.loop(1, NITER - 1)
    def _steps(s):
        ring_step(s, False, False)

    ring_step(NITER - 1, False, True)
    out_wait(ob0, osem0)
    out_wait(ob1, osem1)


@jax.jit
def _run(aspect_flat, class_flat, table):
    mesh = plsc.VectorSubcoreMesh(core_axis_name="c", subcore_axis_name="s")
    k = pl.kernel(
        _body,
        out_type=jax.ShapeDtypeStruct((B, D), jnp.float32),
        mesh=mesh,
        compiler_params=pltpu.CompilerParams(needs_layout_passes=False),
        scratch_types=[
            pltpu.VMEM((RPW * H,), jnp.int32),
            pltpu.VMEM((RPW * H,), jnp.int32),
            pltpu.VMEM((RPW,), jnp.float32),
            pltpu.VMEM((1, D), jnp.float32),
            pltpu.VMEM((IPC, D), jnp.float32),
            pltpu.VMEM((IPC, D), jnp.float32),
            pltpu.VMEM((IPC, D), jnp.float32),
            pltpu.VMEM((IPC, D), jnp.float32),
            pltpu.VMEM((2 * CH, D), jnp.float32),
            pltpu.VMEM((2 * CH, D), jnp.float32),
            pltpu.SemaphoreType.DMA,
            pltpu.SemaphoreType.DMA,
            pltpu.SemaphoreType.DMA,
            pltpu.SemaphoreType.DMA,
            pltpu.SemaphoreType.DMA,
            pltpu.SemaphoreType.DMA,
            pltpu.SemaphoreType.DMA,
            pltpu.SemaphoreType.DMA,
            pltpu.SemaphoreType.DMA,
            pltpu.SemaphoreType.DMA,
        ],
    )
    return k(aspect_flat, class_flat, table)


def kernel(aspect_ids, class_ids, table):
    aspect_flat = aspect_ids.astype(jnp.int32).reshape(-1)
    class_flat = class_ids.astype(jnp.int32).reshape(-1)
    return _run(aspect_flat, class_flat, table.astype(jnp.float32))


# speedup vs baseline: 1.0819x; 1.0049x over previous
import jax
import jax.numpy as jnp
from jax import lax
from jax.experimental import pallas as pl
from jax.experimental.pallas import tpu as pltpu
from jax.experimental.pallas import tpu_sc as plsc

N_CLASSES = 4
D = 128
B = 4096
H = 50

NC = 2
NS = 16
NW = NC * NS
RPW = B // NW
CH = 4
NCHUNK = RPW // CH
IPC = CH * H
SUBS = ((0, 104), (104, 96))
NVD = D // 16
DEPTH = 4
NITER = NCHUNK // DEPTH


def _body(aspect_hbm, class_hbm, table_hbm, out_hbm,
          av, cv, cntv, vinv, t0, rows0, rows1, rows2, rows3, ob0, ob1,
          g0a, g0b, g1a, g1b, g2a, g2b, g3a, g3b, osem0, osem1):
    rows = (rows0, rows1, rows2, rows3)
    gsem = ((g0a, g0b), (g1a, g1b), (g2a, g2b), (g3a, g3b))
    obs = (ob0, ob1)
    osem = (osem0, osem1)

    wid = lax.axis_index("s") * NC + lax.axis_index("c")
    base = wid * RPW
    fbase = base * H

    pltpu.sync_copy(aspect_hbm.at[pl.ds(fbase, RPW * H)], av)
    pltpu.sync_copy(class_hbm.at[pl.ds(fbase, RPW * H)], cv)
    pltpu.sync_copy(table_hbm.at[pl.ds(0, 1)], t0)

    @pl.loop(0, RPW * H // 16, unroll=4)
    def _ids(i):
        off = pl.multiple_of(i * 16, 16)
        a = av[pl.ds(off, 16)]
        c = cv[pl.ds(off, 16)]
        idv = a * N_CLASSES + c
        av[pl.ds(off, 16)] = jnp.where(idv != 0, idv + 1, idv)

    lanes = lax.iota(jnp.int32, 16)

    def take16(vec, lanevec):
        return lax.gather(
            vec, lanevec[:, None],
            lax.GatherDimensionNumbers(offset_dims=(),
                                       collapsed_slice_dims=(0,),
                                       start_index_map=(0,)),
            (1,), mode=lax.GatherScatterMode.PROMISE_IN_BOUNDS)

    def fire(c, rbuf, sems):
        for (soff, slen), sem in zip(SUBS, sems):
            off = pl.multiple_of(c * IPC + soff, 8)
            pltpu.async_copy(
                table_hbm.at[av.at[pl.ds(off, slen)]],
                rbuf.at[pl.ds(soff, slen)], sem)

    def drain_sub(rbuf, sems, j):
        soff, slen = SUBS[j]
        pltpu.make_async_copy(
            table_hbm.at[av.at[pl.ds(soff, slen)]],
            rbuf.at[pl.ds(soff, slen)], sems[j]).wait()

    def accum(c, rbuf, obuf, ohalf, b_lo, b_hi):
        for b in range(b_lo, b_hi):
            r = c * CH + b
            goff = pl.multiple_of((r // 16) * 16, 16)
            lanevec = jnp.broadcast_to(r % 16, (16,))
            padv = take16(cntv[pl.ds(goff, 16)], lanevec)
            invv = take16(vinv[pl.ds(goff, 16)], lanevec)

            def abody(l, accs):
                row = b * H + l * 2
                return tuple(
                    accs[v] + rbuf[row, pl.ds(v * 16, 16)]
                    + rbuf[row + 1, pl.ds(v * 16, 16)]
                    for v in range(NVD))

            accs = lax.fori_loop(
                0, H // 2, abody,
                tuple(jnp.zeros((16,), jnp.float32) for _ in range(NVD)))
            for v in range(NVD):
                t0s = t0[0, pl.ds(v * 16, 16)]
                obuf[ohalf * CH + b, pl.ds(v * 16, 16)] = \
                    (accs[v] - padv * t0s) * invv

    def out_dma(cpair, obuf, sem):
        ooff = pl.multiple_of(base + cpair * 2 * CH, 8)
        pltpu.async_copy(obuf, out_hbm.at[pl.ds(ooff, 2 * CH)], sem)

    def out_wait(obuf, sem):
        pltpu.make_async_copy(obuf, out_hbm.at[pl.ds(0, 2 * CH)], sem).wait()

    for d in range(DEPTH):
        fire(d, rows[d], gsem[d])

    @pl.loop(0, RPW // 16)
    def _cnt(g):
        rowv = g * 16 + lanes

        def cbody(l, acc):
            idv = plsc.load_gather(av, [rowv * H + l])
            return acc + jnp.where(idv == 0, 1.0, 0.0).astype(jnp.float32)

        acc = lax.fori_loop(0, H, cbody, jnp.zeros((16,), jnp.float32))
        goff = pl.multiple_of(g * 16, 16)
        cntv[pl.ds(goff, 16)] = acc
        vinv[pl.ds(goff, 16)] = 1.0 / (jnp.float32(H) - acc)

    def ring_step(s, first, last):
        c0 = s * DEPTH
        for d in range(DEPTH):
            c = c0 + d
            drain_sub(rows[d], gsem[d], 0)
            if not first and d < 2:
                out_wait(obs[d], osem[d])
            accum(c, rows[d], obs[d // 2], d % 2, 0, 2)
            drain_sub(rows[d], gsem[d], 1)
            accum(c, rows[d], obs[d // 2], d % 2, 2, CH)
            if not last:
                fire(c + DEPTH, rows[d], gsem[d])
            if d % 2 == 1:
                out_dma(s * 2 + d // 2, obs[d // 2], osem[d // 2])

    ring_step(0, True, False)

    @pl.loop(1, NITER - 1)
    def _steps(s):
        ring_step(s, False, False)

    ring_step(NITER - 1, False, True)
    out_wait(ob0, osem0)
    out_wait(ob1, osem1)


@jax.jit
def _run(aspect_flat, class_flat, table):
    mesh = plsc.VectorSubcoreMesh(core_axis_name="c", subcore_axis_name="s")
    k = pl.kernel(
        _body,
        out_type=jax.ShapeDtypeStruct((B, D), jnp.float32),
        mesh=mesh,
        compiler_params=pltpu.CompilerParams(needs_layout_passes=False),
        scratch_types=[
            pltpu.VMEM((RPW * H,), jnp.int32),
            pltpu.VMEM((RPW * H,), jnp.int32),
            pltpu.VMEM((RPW,), jnp.float32),
            pltpu.VMEM((RPW,), jnp.float32),
            pltpu.VMEM((1, D), jnp.float32),
            pltpu.VMEM((IPC, D), jnp.float32),
            pltpu.VMEM((IPC, D), jnp.float32),
            pltpu.VMEM((IPC, D), jnp.float32),
            pltpu.VMEM((IPC, D), jnp.float32),
            pltpu.VMEM((2 * CH, D), jnp.float32),
            pltpu.VMEM((2 * CH, D), jnp.float32),
            pltpu.SemaphoreType.DMA,
            pltpu.SemaphoreType.DMA,
            pltpu.SemaphoreType.DMA,
            pltpu.SemaphoreType.DMA,
            pltpu.SemaphoreType.DMA,
            pltpu.SemaphoreType.DMA,
            pltpu.SemaphoreType.DMA,
            pltpu.SemaphoreType.DMA,
            pltpu.SemaphoreType.DMA,
            pltpu.SemaphoreType.DMA,
        ],
    )
    return k(aspect_flat, class_flat, table)


def kernel(aspect_ids, class_ids, table):
    aspect_flat = aspect_ids.astype(jnp.int32).reshape(-1)
    class_flat = class_ids.astype(jnp.int32).reshape(-1)
    return _run(aspect_flat, class_flat, table.astype(jnp.float32))
